# baseline (speedup 1.0000x reference)
import functools

import jax
import jax.numpy as jnp
from jax import lax
from jax.experimental import pallas as pl
from jax.experimental.pallas import tpu as pltpu
from jax.experimental.pallas import tpu_sc as plsc

HIDDEN_DIM = 512
BATCH = 16384
CHUNK = 64


def _make_kernel():
    info = plsc.get_sparse_core_info()
    nw = info.num_cores * info.num_subcores
    b_per_w = BATCH // nw
    n_chunks = b_per_w // CHUNK

    mesh = plsc.VectorSubcoreMesh(core_axis_name="c", subcore_axis_name="s")

    @functools.partial(
        pl.kernel,
        mesh=mesh,
        out_type=jax.ShapeDtypeStruct((BATCH, HIDDEN_DIM), jnp.float32),
        scratch_types=[
            pltpu.VMEM((2, HIDDEN_DIM), jnp.float32),
            pltpu.VMEM((b_per_w + 16,), jnp.int32),
            pltpu.VMEM((CHUNK, HIDDEN_DIM), jnp.float32),
            pltpu.VMEM((CHUNK, HIDDEN_DIM), jnp.float32),
            pltpu.VMEM((CHUNK, HIDDEN_DIM), jnp.float32),
            pltpu.SemaphoreType.DMA,
        ],
    )
    def k(table_hbm, idx_hbm, out_hbm, table_v, idx_v, buf0, buf1, buf2, sem_s):
        wid = lax.axis_index("s") * info.num_cores + lax.axis_index("c")
        base = wid * b_per_w
        pltpu.sync_copy(table_hbm, table_v)
        pltpu.sync_copy(
            idx_hbm.at[pl.ds(base, b_per_w)], idx_v.at[pl.ds(0, b_per_w)]
        )

        bufs = (buf0, buf1, buf2)
        nbuf = len(bufs)
        stores = [None] * n_chunks
        for c in range(n_chunks):
            if c >= nbuf:
                stores[c - nbuf].wait()
            buf = bufs[c % nbuf]

            @plsc.parallel_loop(0, CHUNK, 1, unroll=16)
            def body(r, c=c, buf=buf):
                s = idx_v[pl.ds(c * CHUNK + r, 16)][0]
                for v in range(HIDDEN_DIM // 16):
                    sl = pl.ds(v * 16, 16)
                    buf[r, sl] = table_v[s, sl]

            stores[c] = pltpu.async_copy(
                buf, out_hbm.at[pl.ds(base + c * CHUNK, CHUNK)], sem_s
            )
        for c in range(max(0, n_chunks - nbuf), n_chunks):
            stores[c].wait()

    return k


_lookup = _make_kernel()


def kernel(domain_ids, embed_weight):
    return _lookup(embed_weight, domain_ids.astype(jnp.int32))


# speedup vs baseline: 1.6165x; 1.6165x over previous
import functools

import jax
import jax.numpy as jnp
from jax import lax
from jax.experimental import pallas as pl
from jax.experimental.pallas import tpu as pltpu
from jax.experimental.pallas import tpu_sc as plsc

HIDDEN_DIM = 512
BATCH = 16384
DRAIN = 64


def _make_kernel():
    info = plsc.get_sparse_core_info()
    nw = info.num_cores * info.num_subcores
    b_per_w = BATCH // nw

    mesh = plsc.VectorSubcoreMesh(core_axis_name="c", subcore_axis_name="s")

    @functools.partial(
        pl.kernel,
        mesh=mesh,
        out_type=jax.ShapeDtypeStruct((BATCH, HIDDEN_DIM), jnp.float32),
        scratch_types=[
            pltpu.VMEM((2, HIDDEN_DIM), jnp.float32),
            pltpu.VMEM((b_per_w + 16,), jnp.int32),
            pltpu.VMEM((DRAIN, HIDDEN_DIM), jnp.float32),
            pltpu.SemaphoreType.DMA,
        ],
    )
    def k(table_hbm, idx_hbm, out_hbm, table_v, idx_v, dummy_v, sem_s):
        wid = lax.axis_index("s") * info.num_cores + lax.axis_index("c")
        base = wid * b_per_w
        pltpu.sync_copy(table_hbm, table_v)
        pltpu.sync_copy(
            idx_hbm.at[pl.ds(base, b_per_w)], idx_v.at[pl.ds(0, b_per_w)]
        )

        @plsc.parallel_loop(0, b_per_w, 1, unroll=8)
        def body(r):
            s = idx_v[pl.ds(r, 16)][0]
            pltpu.async_copy(
                table_v.at[pl.ds(s, 1)],
                out_hbm.at[pl.ds(base + r, 1)],
                sem_s,
            )

        for c in range(b_per_w // DRAIN):
            pltpu.make_async_copy(
                out_hbm.at[pl.ds(base + c * DRAIN, DRAIN)], dummy_v, sem_s
            ).wait()

    return k


_lookup = _make_kernel()


def kernel(domain_ids, embed_weight):
    return _lookup(embed_weight, domain_ids.astype(jnp.int32))
